# baseline (speedup 1.0000x reference)
import functools

import jax
import jax.numpy as jnp
from jax.experimental import pallas as pl
from jax.experimental.pallas import tpu as pltpu

IMG_SIZE = 512.0
NCHUNK = 8


def _head_kernel(*refs, tn, w_dim):
    x_refs = refs[:NCHUNK]
    w_ref, b_ref, lab_ref, box_ref, ce_ref = refs[NCHUNK:]
    w = w_ref[...]
    b = b_ref[...]
    for k in range(NCHUNK):
        acc = jnp.dot(w, x_refs[k][0], preferred_element_type=jnp.float32)
        acc = acc + b
        sl = pl.ds(k * tn, tn)
        lab_ref[0, :, sl] = acc[0:20]
        ce_ref[0, :, sl] = acc[24:25]
        d = jnp.exp(acc[20:24])
        hw = k * tn + jax.lax.broadcasted_iota(jnp.int32, (1, tn), 1)
        stride = IMG_SIZE / w_dim
        cy = ((hw // w_dim).astype(jnp.float32) + 0.5) * stride
        cx = ((hw % w_dim).astype(jnp.float32) + 0.5) * stride
        box_ref[0, :, sl] = jnp.concatenate(
            [cx - d[0:1], cy - d[1:2], cx + d[2:3], cy + d[3:4]], axis=0)


def kernel(x, Wc, bc, Wb, bb, Wce, bce):
    B, C, H, W = x.shape
    HW = H * W
    TN = HW // NCHUNK
    nclasses = Wc.shape[0]

    xf = x.reshape(B, C, HW)
    Wf = jnp.concatenate([Wc, Wb, Wce], axis=0)
    bf = jnp.concatenate([bc, bb, bce], axis=0)[:, None]

    def x_spec(k):
        return pl.BlockSpec((1, C, TN), lambda i, k=k: (i, 0, k))

    labels, boxes, ctr = pl.pallas_call(
        functools.partial(_head_kernel, tn=TN, w_dim=W),
        grid=(B,),
        in_specs=[x_spec(k) for k in range(NCHUNK)] + [
            pl.BlockSpec((nclasses + 5, C), lambda i: (0, 0)),
            pl.BlockSpec((nclasses + 5, 1), lambda i: (0, 0)),
        ],
        out_specs=[
            pl.BlockSpec((1, nclasses, HW), lambda i: (i, 0, 0)),
            pl.BlockSpec((1, 4, HW), lambda i: (i, 0, 0)),
            pl.BlockSpec((1, 1, HW), lambda i: (i, 0, 0)),
        ],
        out_shape=[
            jax.ShapeDtypeStruct((B, nclasses, HW), jnp.float32),
            jax.ShapeDtypeStruct((B, 4, HW), jnp.float32),
            jax.ShapeDtypeStruct((B, 1, HW), jnp.float32),
        ],
        compiler_params=pltpu.CompilerParams(
            dimension_semantics=("parallel",)),
    )(*([xf] * NCHUNK), Wf, bf)

    return (labels.reshape(B, nclasses, H, W),
            boxes.reshape(B, 4, H, W),
            ctr.reshape(B, 1, H, W))
